# Initial kernel scaffold; baseline (speedup 1.0000x reference)
import jax
import jax.numpy as jnp
from jax.experimental import pallas as pl


def kernel(x_user, x_item, edge_index_u2i, edge_index_i2u, edge_attr_u2i, edge_attr_i2u, Wl, bl, Wr, br, We, be, gamma, beta):
    raise NotImplementedError("write your pallas kernel here")


# speedup vs baseline: 3.2082x; 3.2082x over previous
import functools

import jax
import jax.numpy as jnp
from jax import lax
from jax.experimental import pallas as pl
from jax.experimental.pallas import tpu as pltpu
from jax.experimental.pallas import tpu_sc as plsc

NUSER = 10000
NITEM = 10000
NE = 320000
DD = 128
NLAYERS = 3
NC = 2
NS = 16
NW = NC * NS
CHUNK = 80
EPW = NE // NW
CPW = EPW // CHUNK
RPT = 640
NPAD = NS * RPT

_MESH = plsc.VectorSubcoreMesh(core_axis_name="c", subcore_axis_name="s")


def _sc_attr_body(ea0, d0, ea1, d1, z16, out0, out1, didx, rowbuf, sacc):
    c = lax.axis_index("c")
    s = lax.axis_index("s")
    wid = s * NC + c
    r0 = s * RPT

    def run(ea_hbm, dst_hbm, out_hbm):
        pltpu.sync_copy(z16, sacc.at[pl.ds(r0, RPT)])
        plsc.subcore_barrier()

        def body(j, carry):
            base = wid * EPW + j * CHUNK
            pltpu.sync_copy(dst_hbm.at[pl.ds(base, CHUNK)], didx)
            pltpu.sync_copy(ea_hbm.at[pl.ds(base, CHUNK)], rowbuf)
            pltpu.sync_copy(rowbuf, sacc.at[didx], add=True)
            return carry

        lax.fori_loop(0, CPW, body, 0, unroll=False)
        plsc.subcore_barrier()
        pltpu.sync_copy(sacc.at[pl.ds(r0, RPT)], out_hbm.at[c, pl.ds(r0, RPT)])
        plsc.subcore_barrier()

    run(ea0, d0, out0)
    run(ea1, d1, out1)


def _sc_layer_body(hu, hi, su2i, du2i, si2u, di2u, zrows,
                   out_i, out_u, sidx, didx, rowbuf, sacc, sem):
    c = lax.axis_index("c")
    s = lax.axis_index("s")
    wid = s * NC + c
    r0 = s * RPT

    def run(h_hbm, src_hbm, dst_hbm, out_hbm):
        pltpu.sync_copy(zrows, sacc.at[pl.ds(r0, RPT)])
        plsc.subcore_barrier()

        def body(j, carry):
            base = wid * EPW + j * CHUNK
            pltpu.sync_copy(src_hbm.at[pl.ds(base, CHUNK)], sidx)
            pltpu.sync_copy(dst_hbm.at[pl.ds(base, CHUNK)], didx)
            pltpu.async_copy(h_hbm.at[sidx], rowbuf, sem).wait()
            pltpu.sync_copy(rowbuf, sacc.at[didx], add=True)
            return carry

        lax.fori_loop(0, CPW, body, 0, unroll=False)
        plsc.subcore_barrier()
        pltpu.sync_copy(sacc.at[pl.ds(r0, RPT)], out_hbm.at[c, pl.ds(r0, RPT)])
        plsc.subcore_barrier()

    run(hu, su2i, du2i, out_i)
    run(hi, si2u, di2u, out_u)


_sc_attr = pl.kernel(
    _sc_attr_body,
    out_type=(
        jax.ShapeDtypeStruct((NC, NPAD, DD), jnp.float32),
        jax.ShapeDtypeStruct((NC, NPAD, DD), jnp.float32),
    ),
    mesh=_MESH,
    scratch_types=[
        pltpu.VMEM((CHUNK,), jnp.int32),
        pltpu.VMEM((CHUNK, DD), jnp.float32),
        pltpu.VMEM_SHARED((NPAD, DD), jnp.float32),
    ],
)

_sc_layer = pl.kernel(
    _sc_layer_body,
    out_type=(
        jax.ShapeDtypeStruct((NC, NPAD, DD), jnp.float32),
        jax.ShapeDtypeStruct((NC, NPAD, DD), jnp.float32),
    ),
    mesh=_MESH,
    scratch_types=[
        pltpu.VMEM((CHUNK,), jnp.int32),
        pltpu.VMEM((CHUNK,), jnp.int32),
        pltpu.VMEM((CHUNK, DD), jnp.float32),
        pltpu.VMEM_SHARED((NPAD, DD), jnp.float32),
        pltpu.SemaphoreType.DMA,
    ],
)


def _tc_layer_kern(pu, pi, au, ai, hu, hi, weu, wei,
                   wlu, wru, wli, wri, bb, gam, bet, ou, oi):
    def side(p, a, h, we, wl, wr, row):
        ssum = p[0] + p[1]
        a16 = (a[0] + a[1])[:, :16]
        cnt = a16[:, 15:16]
        se = jnp.dot(a16, we[...], preferred_element_type=jnp.float32)
        agg = (ssum + se) / jnp.maximum(cnt, 1.0)
        out = (jnp.dot(agg, wl[...], preferred_element_type=jnp.float32)
               + jnp.dot(h[...], wr[...], preferred_element_type=jnp.float32)
               + bb[row:row + 1])
        m = jnp.mean(out, axis=-1, keepdims=True)
        var = jnp.mean((out - m) ** 2, axis=-1, keepdims=True)
        return (out - m) * lax.rsqrt(var + 1e-5) * gam[row:row + 1] + bet[row:row + 1]

    ou[...] = side(pu, au, hu, weu, wlu, wru, 0)
    oi[...] = side(pi, ai, hi, wei, wli, wri, 1)


def _tc_layer(pu, pi, au, ai, hu, hi, weu, wei, wlu, wru, wli, wri, bb, gam, bet):
    bt = 1000
    grid = (NUSER // bt,)
    full2 = lambda shape: pl.BlockSpec(shape, lambda b: (0, 0))
    out = pl.pallas_call(
        _tc_layer_kern,
        grid=grid,
        in_specs=[
            pl.BlockSpec((NC, bt, DD), lambda b: (0, b, 0)),
            pl.BlockSpec((NC, bt, DD), lambda b: (0, b, 0)),
            pl.BlockSpec((NC, bt, DD), lambda b: (0, b, 0)),
            pl.BlockSpec((NC, bt, DD), lambda b: (0, b, 0)),
            pl.BlockSpec((bt, DD), lambda b: (b, 0)),
            pl.BlockSpec((bt, DD), lambda b: (b, 0)),
            full2((16, DD)), full2((16, DD)),
            full2((DD, DD)), full2((DD, DD)), full2((DD, DD)), full2((DD, DD)),
            full2((2, DD)), full2((2, DD)), full2((2, DD)),
        ],
        out_specs=[
            pl.BlockSpec((bt, DD), lambda b: (b, 0)),
            pl.BlockSpec((bt, DD), lambda b: (b, 0)),
        ],
        out_shape=[
            jax.ShapeDtypeStruct((NUSER, DD), jnp.float32),
            jax.ShapeDtypeStruct((NITEM, DD), jnp.float32),
        ],
    )(pu, pi, au, ai, hu, hi, weu, wei, wlu, wru, wli, wri, bb, gam, bet)
    return out


def kernel(x_user, x_item, edge_index_u2i, edge_index_i2u,
           edge_attr_u2i, edge_attr_i2u, Wl, bl, Wr, br, We, be, gamma, beta):
    f32 = jnp.float32
    su2i = edge_index_u2i[0].astype(jnp.int32)
    du2i = edge_index_u2i[1].astype(jnp.int32)
    si2u = edge_index_i2u[0].astype(jnp.int32)
    di2u = edge_index_i2u[1].astype(jnp.int32)

    def pad128(ea):
        z = jnp.zeros((NE, 6), f32)
        o = jnp.ones((NE, 1), f32)
        z2 = jnp.zeros((NE, DD - 16), f32)
        return jnp.concatenate([ea.astype(f32), z, o, z2], axis=1)

    ea0 = pad128(edge_attr_u2i)
    ea1 = pad128(edge_attr_i2u)

    def padWe(e):
        return jnp.concatenate(
            [We[e].astype(f32), jnp.zeros((6, DD), f32), be[e][None].astype(f32)],
            axis=0)

    we_i = padWe(0)
    we_u = padWe(1)

    z16 = jnp.zeros((RPT, DD), f32)
    zrows = jnp.zeros((RPT, DD), f32)

    ai16, au16 = _sc_attr(ea0, du2i, ea1, di2u, z16)

    hu = x_user.astype(f32)
    hi = x_item.astype(f32)
    gam = gamma.astype(f32)
    bet = beta.astype(f32)

    for l in range(NLAYERS):
        pi, pu = _sc_layer(hu, hi, su2i, du2i, si2u, di2u, zrows)
        bb = jnp.stack([bl[1, l] + br[1, l], bl[0, l] + br[0, l]], axis=0).astype(f32)
        hu, hi = _tc_layer(pu, pi, au16, ai16, hu, hi, we_u, we_i,
                           Wl[1, l], Wr[1, l], Wl[0, l], Wr[0, l], bb, gam, bet)
    return jnp.concatenate([hu, hi], axis=0)


# speedup vs baseline: 6.9180x; 2.1564x over previous
import functools

import jax
import jax.numpy as jnp
from jax import lax
from jax.experimental import pallas as pl
from jax.experimental.pallas import tpu as pltpu
from jax.experimental.pallas import tpu_sc as plsc

NUSER = 10000
NITEM = 10000
NE = 320000
DD = 128
NLAYERS = 3
NC = 2
NS = 16
NW = NC * NS
CHUNK = 80
EPW = NE // NW
CPW = EPW // CHUNK
RPT = 640
NPAD = NS * RPT

_MESH = plsc.VectorSubcoreMesh(core_axis_name="c", subcore_axis_name="s")


def _pipelined_phase(indirect, h_hbm, src_hbm, dst_hbm, out_hbm,
                     c, s, wid, sall, didx2, rb0, rb1, sacc, zrows,
                     semi0, semi1, semg0, semg1):
    r0 = s * RPT
    pltpu.sync_copy(zrows, sacc.at[pl.ds(r0, RPT)])
    if indirect:
        pltpu.sync_copy(src_hbm.at[pl.ds(wid * EPW, EPW)], sall)
    plsc.subcore_barrier()

    semi = (semi0, semi1)
    semg = (semg0, semg1)
    rb = (rb0, rb1)

    def idx_start(j, slot):
        base = wid * EPW + j * CHUNK
        pltpu.async_copy(dst_hbm.at[pl.ds(base, CHUNK)], didx2.at[slot],
                         semi[slot])

    def gather_start(j, slot):
        if indirect:
            srcref = h_hbm.at[sall.at[pl.ds(j * CHUNK, CHUNK)]]
        else:
            srcref = h_hbm.at[pl.ds(wid * EPW + j * CHUNK, CHUNK)]
        pltpu.async_copy(srcref, rb[slot], semg[slot])

    def drain(j, slot, prefetch):
        pltpu.make_async_copy(h_hbm.at[pl.ds(0, CHUNK)] if not indirect
                              else h_hbm.at[sall.at[pl.ds(0, CHUNK)]],
                              rb[slot], semg[slot]).wait()
        pltpu.make_async_copy(dst_hbm.at[pl.ds(0, CHUNK)], didx2.at[slot],
                              semi[slot]).wait()
        pltpu.sync_copy(rb[slot], sacc.at[didx2.at[slot]], add=True)
        if prefetch:
            @pl.when(j + 2 < CPW)
            def _():
                idx_start(j + 2, slot)

    idx_start(0, 0)
    idx_start(1, 1)
    gather_start(0, 0)

    def body(j2, carry):
        j = j2 * 2
        gather_start(j + 1, 1)
        drain(j, 0, True)
        gather_start(j + 2, 0)
        drain(j + 1, 1, True)
        return carry

    lax.fori_loop(0, (CPW - 1) // 2, body, 0, unroll=False)
    drain(CPW - 1, 0, False)

    plsc.subcore_barrier()
    pltpu.sync_copy(sacc.at[pl.ds(r0, RPT)], out_hbm.at[c, pl.ds(r0, RPT)])
    plsc.subcore_barrier()


def _sc_attr_body(ea0, d0, ea1, d1, z16, out0, out1,
                  sall, didx2, rb0, rb1, sacc,
                  semi0, semi1, semg0, semg1):
    c = lax.axis_index("c")
    s = lax.axis_index("s")
    wid = s * NC + c
    _pipelined_phase(False, ea0, None, d0, out0, c, s, wid, sall, didx2,
                     rb0, rb1, sacc, z16, semi0, semi1, semg0, semg1)
    _pipelined_phase(False, ea1, None, d1, out1, c, s, wid, sall, didx2,
                     rb0, rb1, sacc, z16, semi0, semi1, semg0, semg1)


def _sc_layer_body(hu, hi, su2i, du2i, si2u, di2u, zrows,
                   out_i, out_u, sall, didx2, rb0, rb1, sacc,
                   semi0, semi1, semg0, semg1):
    c = lax.axis_index("c")
    s = lax.axis_index("s")
    wid = s * NC + c
    _pipelined_phase(True, hu, su2i, du2i, out_i, c, s, wid, sall, didx2,
                     rb0, rb1, sacc, zrows, semi0, semi1, semg0, semg1)
    _pipelined_phase(True, hi, si2u, di2u, out_u, c, s, wid, sall, didx2,
                     rb0, rb1, sacc, zrows, semi0, semi1, semg0, semg1)


_sc_attr = pl.kernel(
    _sc_attr_body,
    out_type=(
        jax.ShapeDtypeStruct((NC, NPAD, DD), jnp.float32),
        jax.ShapeDtypeStruct((NC, NPAD, DD), jnp.float32),
    ),
    mesh=_MESH,
    scratch_types=[
        pltpu.VMEM((EPW,), jnp.int32),
        pltpu.VMEM((2, CHUNK), jnp.int32),
        pltpu.VMEM((CHUNK, DD), jnp.float32),
        pltpu.VMEM((CHUNK, DD), jnp.float32),
        pltpu.VMEM_SHARED((NPAD, DD), jnp.float32),
        pltpu.SemaphoreType.DMA,
        pltpu.SemaphoreType.DMA,
        pltpu.SemaphoreType.DMA,
        pltpu.SemaphoreType.DMA,
    ],
)

_sc_layer = pl.kernel(
    _sc_layer_body,
    out_type=(
        jax.ShapeDtypeStruct((NC, NPAD, DD), jnp.float32),
        jax.ShapeDtypeStruct((NC, NPAD, DD), jnp.float32),
    ),
    mesh=_MESH,
    scratch_types=[
        pltpu.VMEM((EPW,), jnp.int32),
        pltpu.VMEM((2, CHUNK), jnp.int32),
        pltpu.VMEM((CHUNK, DD), jnp.float32),
        pltpu.VMEM((CHUNK, DD), jnp.float32),
        pltpu.VMEM_SHARED((NPAD, DD), jnp.float32),
        pltpu.SemaphoreType.DMA,
        pltpu.SemaphoreType.DMA,
        pltpu.SemaphoreType.DMA,
        pltpu.SemaphoreType.DMA,
    ],
)


def _tc_layer_kern(pu, pi, au, ai, hu, hi, weu, wei,
                   wlu, wru, wli, wri, bb, gam, bet, ou, oi):
    def side(p, a, h, we, wl, wr, row):
        ssum = p[0] + p[1]
        a16 = (a[0] + a[1])[:, :16]
        cnt = a16[:, 15:16]
        se = jnp.dot(a16, we[...], preferred_element_type=jnp.float32)
        agg = (ssum + se) / jnp.maximum(cnt, 1.0)
        out = (jnp.dot(agg, wl[...], preferred_element_type=jnp.float32)
               + jnp.dot(h[...], wr[...], preferred_element_type=jnp.float32)
               + bb[row:row + 1])
        m = jnp.mean(out, axis=-1, keepdims=True)
        var = jnp.mean((out - m) ** 2, axis=-1, keepdims=True)
        return (out - m) * lax.rsqrt(var + 1e-5) * gam[row:row + 1] + bet[row:row + 1]

    ou[...] = side(pu, au, hu, weu, wlu, wru, 0)
    oi[...] = side(pi, ai, hi, wei, wli, wri, 1)


def _tc_layer(pu, pi, au, ai, hu, hi, weu, wei, wlu, wru, wli, wri, bb, gam, bet):
    bt = 1000
    grid = (NUSER // bt,)
    full2 = lambda shape: pl.BlockSpec(shape, lambda b: (0, 0))
    out = pl.pallas_call(
        _tc_layer_kern,
        grid=grid,
        in_specs=[
            pl.BlockSpec((NC, bt, DD), lambda b: (0, b, 0)),
            pl.BlockSpec((NC, bt, DD), lambda b: (0, b, 0)),
            pl.BlockSpec((NC, bt, DD), lambda b: (0, b, 0)),
            pl.BlockSpec((NC, bt, DD), lambda b: (0, b, 0)),
            pl.BlockSpec((bt, DD), lambda b: (b, 0)),
            pl.BlockSpec((bt, DD), lambda b: (b, 0)),
            full2((16, DD)), full2((16, DD)),
            full2((DD, DD)), full2((DD, DD)), full2((DD, DD)), full2((DD, DD)),
            full2((2, DD)), full2((2, DD)), full2((2, DD)),
        ],
        out_specs=[
            pl.BlockSpec((bt, DD), lambda b: (b, 0)),
            pl.BlockSpec((bt, DD), lambda b: (b, 0)),
        ],
        out_shape=[
            jax.ShapeDtypeStruct((NUSER, DD), jnp.float32),
            jax.ShapeDtypeStruct((NITEM, DD), jnp.float32),
        ],
    )(pu, pi, au, ai, hu, hi, weu, wei, wlu, wru, wli, wri, bb, gam, bet)
    return out


def kernel(x_user, x_item, edge_index_u2i, edge_index_i2u,
           edge_attr_u2i, edge_attr_i2u, Wl, bl, Wr, br, We, be, gamma, beta):
    f32 = jnp.float32
    su2i = edge_index_u2i[0].astype(jnp.int32)
    du2i = edge_index_u2i[1].astype(jnp.int32)
    si2u = edge_index_i2u[0].astype(jnp.int32)
    di2u = edge_index_i2u[1].astype(jnp.int32)

    def pad128(ea):
        z = jnp.zeros((NE, 6), f32)
        o = jnp.ones((NE, 1), f32)
        z2 = jnp.zeros((NE, DD - 16), f32)
        return jnp.concatenate([ea.astype(f32), z, o, z2], axis=1)

    ea0 = pad128(edge_attr_u2i)
    ea1 = pad128(edge_attr_i2u)

    def padWe(e):
        return jnp.concatenate(
            [We[e].astype(f32), jnp.zeros((6, DD), f32), be[e][None].astype(f32)],
            axis=0)

    we_i = padWe(0)
    we_u = padWe(1)

    z16 = jnp.zeros((RPT, DD), f32)
    zrows = jnp.zeros((RPT, DD), f32)

    ai16, au16 = _sc_attr(ea0, du2i, ea1, di2u, z16)

    hu = x_user.astype(f32)
    hi = x_item.astype(f32)
    gam = gamma.astype(f32)
    bet = beta.astype(f32)

    for l in range(NLAYERS):
        pi, pu = _sc_layer(hu, hi, su2i, du2i, si2u, di2u, zrows)
        bb = jnp.stack([bl[1, l] + br[1, l], bl[0, l] + br[0, l]], axis=0).astype(f32)
        hu, hi = _tc_layer(pu, pi, au16, ai16, hu, hi, we_u, we_i,
                           Wl[1, l], Wr[1, l], Wl[0, l], Wr[0, l], bb, gam, bet)
    return jnp.concatenate([hu, hi], axis=0)


# speedup vs baseline: 7.4215x; 1.0728x over previous
import functools

import jax
import jax.numpy as jnp
from jax import lax
from jax.experimental import pallas as pl
from jax.experimental.pallas import tpu as pltpu
from jax.experimental.pallas import tpu_sc as plsc

NUSER = 10000
NITEM = 10000
NE = 320000
DD = 128
NLAYERS = 3
NC = 2
NS = 16
NW = NC * NS
CHUNK = 128
EPW = NE // NW
NCH = EPW // CHUNK
TAIL = EPW - NCH * CHUNK
NRB = 2
NIX = 4
RPT = 640
NPAD = NS * RPT

_MESH = plsc.VectorSubcoreMesh(core_axis_name="c", subcore_axis_name="s")


def _pipelined_phase(indirect, h_hbm, src_hbm, dst_hbm, out_hbm,
                     c, s, wid, sall, didx2, rbs, rbt, didxt, sacc, zrows,
                     semis, semgs, semt):
    r0 = s * RPT
    pltpu.sync_copy(zrows, sacc.at[pl.ds(r0, RPT)])
    if indirect:
        pltpu.sync_copy(src_hbm.at[pl.ds(wid * EPW, EPW)], sall)
    plsc.subcore_barrier()

    def idx_start(j, islot):
        base = wid * EPW + j * CHUNK
        pltpu.async_copy(dst_hbm.at[pl.ds(base, CHUNK)], didx2.at[islot],
                         semis[islot])

    def gather_start(j, rslot):
        if indirect:
            srcref = h_hbm.at[sall.at[pl.ds(j * CHUNK, CHUNK)]]
        else:
            srcref = h_hbm.at[pl.ds(wid * EPW + j * CHUNK, CHUNK)]
        pltpu.async_copy(srcref, rbs[rslot], semgs[rslot])

    def drain(j, rslot, islot):
        pltpu.make_async_copy(h_hbm.at[pl.ds(0, CHUNK)] if not indirect
                              else h_hbm.at[sall.at[pl.ds(0, CHUNK)]],
                              rbs[rslot], semgs[rslot]).wait()
        pltpu.make_async_copy(dst_hbm.at[pl.ds(0, CHUNK)], didx2.at[islot],
                              semis[islot]).wait()
        pltpu.sync_copy(rbs[rslot], sacc.at[didx2.at[islot]], add=True)

    def sub(j, rslot, islot):
        @pl.when(j + 1 < NCH)
        def _():
            gather_start(j + 1, 1 - rslot)
        drain(j, rslot, islot)
        @pl.when(j + NIX < NCH)
        def _():
            idx_start(j + NIX, islot)

    for t in range(NIX):
        idx_start(t, t)
    gather_start(0, 0)

    def body(g, carry):
        j0 = g * NIX
        for t in range(NIX):
            sub(j0 + t, t % NRB, t)
        return carry

    ngroups = NCH // NIX
    lax.fori_loop(0, ngroups, body, 0, unroll=False)
    for t in range(NCH - ngroups * NIX):
        sub(ngroups * NIX + t, t % NRB, t)

    tb = wid * EPW + NCH * CHUNK
    pltpu.sync_copy(dst_hbm.at[pl.ds(tb, TAIL)], didxt)
    if indirect:
        pltpu.async_copy(h_hbm.at[sall.at[pl.ds(NCH * CHUNK, TAIL)]],
                         rbt, semt).wait()
    else:
        pltpu.async_copy(h_hbm.at[pl.ds(tb, TAIL)], rbt, semt).wait()
    pltpu.sync_copy(rbt, sacc.at[didxt], add=True)

    plsc.subcore_barrier()
    pltpu.sync_copy(sacc.at[pl.ds(r0, RPT)], out_hbm.at[c, pl.ds(r0, RPT)])
    plsc.subcore_barrier()


def _sc_attr_body(ea0, d0, ea1, d1, z16, out0, out1,
                  sall, didx2, rb0, rb1, rbt, didxt, sacc,
                  semi0, semi1, semi2, semi3, semg0, semg1, semt):
    c = lax.axis_index("c")
    s = lax.axis_index("s")
    wid = s * NC + c
    rbs = (rb0, rb1)
    semis = (semi0, semi1, semi2, semi3)
    semgs = (semg0, semg1)
    _pipelined_phase(False, ea0, None, d0, out0, c, s, wid, sall, didx2,
                     rbs, rbt, didxt, sacc, z16, semis, semgs, semt)
    _pipelined_phase(False, ea1, None, d1, out1, c, s, wid, sall, didx2,
                     rbs, rbt, didxt, sacc, z16, semis, semgs, semt)


def _sc_layer_body(hu, hi, su2i, du2i, si2u, di2u, zrows,
                   out_i, out_u, sall, didx2, rb0, rb1, rbt,
                   didxt, sacc, semi0, semi1, semi2, semi3,
                   semg0, semg1, semt):
    c = lax.axis_index("c")
    s = lax.axis_index("s")
    wid = s * NC + c
    rbs = (rb0, rb1)
    semis = (semi0, semi1, semi2, semi3)
    semgs = (semg0, semg1)
    _pipelined_phase(True, hu, su2i, du2i, out_i, c, s, wid, sall, didx2,
                     rbs, rbt, didxt, sacc, zrows, semis, semgs, semt)
    _pipelined_phase(True, hi, si2u, di2u, out_u, c, s, wid, sall, didx2,
                     rbs, rbt, didxt, sacc, zrows, semis, semgs, semt)


_sc_attr = pl.kernel(
    _sc_attr_body,
    out_type=(
        jax.ShapeDtypeStruct((NC, NPAD, DD), jnp.float32),
        jax.ShapeDtypeStruct((NC, NPAD, DD), jnp.float32),
    ),
    mesh=_MESH,
    scratch_types=[
        pltpu.VMEM((EPW,), jnp.int32),
        pltpu.VMEM((NIX, CHUNK), jnp.int32),
        pltpu.VMEM((CHUNK, DD), jnp.float32),
        pltpu.VMEM((CHUNK, DD), jnp.float32),
        pltpu.VMEM((TAIL, DD), jnp.float32),
        pltpu.VMEM((TAIL,), jnp.int32),
        pltpu.VMEM_SHARED((NPAD, DD), jnp.float32),
        pltpu.SemaphoreType.DMA,
        pltpu.SemaphoreType.DMA,
        pltpu.SemaphoreType.DMA,
        pltpu.SemaphoreType.DMA,
        pltpu.SemaphoreType.DMA,
        pltpu.SemaphoreType.DMA,
        pltpu.SemaphoreType.DMA,
    ],
)

_sc_layer = pl.kernel(
    _sc_layer_body,
    out_type=(
        jax.ShapeDtypeStruct((NC, NPAD, DD), jnp.float32),
        jax.ShapeDtypeStruct((NC, NPAD, DD), jnp.float32),
    ),
    mesh=_MESH,
    scratch_types=[
        pltpu.VMEM((EPW,), jnp.int32),
        pltpu.VMEM((NIX, CHUNK), jnp.int32),
        pltpu.VMEM((CHUNK, DD), jnp.float32),
        pltpu.VMEM((CHUNK, DD), jnp.float32),
        pltpu.VMEM((TAIL, DD), jnp.float32),
        pltpu.VMEM((TAIL,), jnp.int32),
        pltpu.VMEM_SHARED((NPAD, DD), jnp.float32),
        pltpu.SemaphoreType.DMA,
        pltpu.SemaphoreType.DMA,
        pltpu.SemaphoreType.DMA,
        pltpu.SemaphoreType.DMA,
        pltpu.SemaphoreType.DMA,
        pltpu.SemaphoreType.DMA,
        pltpu.SemaphoreType.DMA,
    ],
)


def _tc_layer_kern(pu, pi, au, ai, hu, hi, weu, wei,
                   wlu, wru, wli, wri, bb, gam, bet, ou, oi):
    def side(p, a, h, we, wl, wr, row):
        ssum = p[0] + p[1]
        a16 = (a[0] + a[1])[:, :16]
        cnt = a16[:, 15:16]
        se = jnp.dot(a16, we[...], preferred_element_type=jnp.float32)
        agg = (ssum + se) / jnp.maximum(cnt, 1.0)
        out = (jnp.dot(agg, wl[...], preferred_element_type=jnp.float32)
               + jnp.dot(h[...], wr[...], preferred_element_type=jnp.float32)
               + bb[row:row + 1])
        m = jnp.mean(out, axis=-1, keepdims=True)
        var = jnp.mean((out - m) ** 2, axis=-1, keepdims=True)
        return (out - m) * lax.rsqrt(var + 1e-5) * gam[row:row + 1] + bet[row:row + 1]

    ou[...] = side(pu, au, hu, weu, wlu, wru, 0)
    oi[...] = side(pi, ai, hi, wei, wli, wri, 1)


def _tc_layer(pu, pi, au, ai, hu, hi, weu, wei, wlu, wru, wli, wri, bb, gam, bet):
    bt = 1000
    grid = (NUSER // bt,)
    full2 = lambda shape: pl.BlockSpec(shape, lambda b: (0, 0))
    out = pl.pallas_call(
        _tc_layer_kern,
        grid=grid,
        in_specs=[
            pl.BlockSpec((NC, bt, DD), lambda b: (0, b, 0)),
            pl.BlockSpec((NC, bt, DD), lambda b: (0, b, 0)),
            pl.BlockSpec((NC, bt, DD), lambda b: (0, b, 0)),
            pl.BlockSpec((NC, bt, DD), lambda b: (0, b, 0)),
            pl.BlockSpec((bt, DD), lambda b: (b, 0)),
            pl.BlockSpec((bt, DD), lambda b: (b, 0)),
            full2((16, DD)), full2((16, DD)),
            full2((DD, DD)), full2((DD, DD)), full2((DD, DD)), full2((DD, DD)),
            full2((2, DD)), full2((2, DD)), full2((2, DD)),
        ],
        out_specs=[
            pl.BlockSpec((bt, DD), lambda b: (b, 0)),
            pl.BlockSpec((bt, DD), lambda b: (b, 0)),
        ],
        out_shape=[
            jax.ShapeDtypeStruct((NUSER, DD), jnp.float32),
            jax.ShapeDtypeStruct((NITEM, DD), jnp.float32),
        ],
    )(pu, pi, au, ai, hu, hi, weu, wei, wlu, wru, wli, wri, bb, gam, bet)
    return out


def kernel(x_user, x_item, edge_index_u2i, edge_index_i2u,
           edge_attr_u2i, edge_attr_i2u, Wl, bl, Wr, br, We, be, gamma, beta):
    f32 = jnp.float32
    su2i = edge_index_u2i[0].astype(jnp.int32)
    du2i = edge_index_u2i[1].astype(jnp.int32)
    si2u = edge_index_i2u[0].astype(jnp.int32)
    di2u = edge_index_i2u[1].astype(jnp.int32)

    def pad128(ea):
        z = jnp.zeros((NE, 6), f32)
        o = jnp.ones((NE, 1), f32)
        z2 = jnp.zeros((NE, DD - 16), f32)
        return jnp.concatenate([ea.astype(f32), z, o, z2], axis=1)

    ea0 = pad128(edge_attr_u2i)
    ea1 = pad128(edge_attr_i2u)

    def padWe(e):
        return jnp.concatenate(
            [We[e].astype(f32), jnp.zeros((6, DD), f32), be[e][None].astype(f32)],
            axis=0)

    we_i = padWe(0)
    we_u = padWe(1)

    z16 = jnp.zeros((RPT, DD), f32)
    zrows = jnp.zeros((RPT, DD), f32)

    ai16, au16 = _sc_attr(ea0, du2i, ea1, di2u, z16)

    hu = x_user.astype(f32)
    hi = x_item.astype(f32)
    gam = gamma.astype(f32)
    bet = beta.astype(f32)

    for l in range(NLAYERS):
        pi, pu = _sc_layer(hu, hi, su2i, du2i, si2u, di2u, zrows)
        bb = jnp.stack([bl[1, l] + br[1, l], bl[0, l] + br[0, l]], axis=0).astype(f32)
        hu, hi = _tc_layer(pu, pi, au16, ai16, hu, hi, we_u, we_i,
                           Wl[1, l], Wr[1, l], Wl[0, l], Wr[0, l], bb, gam, bet)
    return jnp.concatenate([hu, hi], axis=0)


# speedup vs baseline: 7.6098x; 1.0254x over previous
import functools

import jax
import jax.numpy as jnp
from jax import lax
from jax.experimental import pallas as pl
from jax.experimental.pallas import tpu as pltpu
from jax.experimental.pallas import tpu_sc as plsc

NUSER = 10000
NITEM = 10000
NE = 320000
DD = 128
NLAYERS = 3
NC = 2
NS = 16
NW = NC * NS
CHUNK = 80
EPW = NE // NW
TOT = EPW // CHUNK
NRB = 4
NIX = 8
RPT = 640
NPAD = NS * RPT

_MESH = plsc.VectorSubcoreMesh(core_axis_name="c", subcore_axis_name="s")


def _pipelined_phase(indirect, h_hbm, src_hbm, dst_hbm, out_hbm,
                     c, s, wid, sidx2, didx2, rbs, sacc, zrows,
                     semg, sems, semsi, semdi):
    r0 = s * RPT
    pltpu.sync_copy(zrows, sacc.at[pl.ds(r0, RPT)])
    plsc.subcore_barrier()

    def idx_start(j, i):
        base = wid * EPW + j * CHUNK
        if indirect:
            pltpu.async_copy(src_hbm.at[pl.ds(base, CHUNK)], sidx2.at[i],
                             semsi.at[i])
        pltpu.async_copy(dst_hbm.at[pl.ds(base, CHUNK)], didx2.at[i],
                         semdi.at[i])

    def gather_start(j, r, i):
        if indirect:
            pltpu.make_async_copy(src_hbm.at[pl.ds(0, CHUNK)], sidx2.at[i],
                                  semsi.at[i]).wait()
            pltpu.async_copy(h_hbm.at[sidx2.at[i]], rbs.at[r], semg.at[r])
        else:
            base = wid * EPW + j * CHUNK
            pltpu.async_copy(h_hbm.at[pl.ds(base, CHUNK)], rbs.at[r],
                             semg.at[r])

    def scatter_wait(r, i):
        pltpu.make_async_copy(rbs.at[r], sacc.at[didx2.at[i]],
                              sems.at[r]).wait()

    def sub(j, t):
        rcur = t % NRB
        icur = t % NIX
        rnext = (t + 2) % NRB
        iold = (t + NIX - 2) % NIX
        inext = (t + 2) % NIX
        def _when(cond, fn):
            if isinstance(cond, bool):
                if cond:
                    fn()
            else:
                pl.when(cond)(fn)
        if isinstance(j, int):
            _when(j >= 2, lambda: scatter_wait(rnext, iold))
            _when(j >= 2 and j + NIX - 2 < TOT,
                  lambda: idx_start(j + NIX - 2, iold))
            _when(j + 2 < TOT, lambda: gather_start(j + 2, rnext, inext))
        else:
            _when(j >= 2, lambda: scatter_wait(rnext, iold))
            _when(jnp.logical_and(j >= 2, j + NIX - 2 < TOT),
                  lambda: idx_start(j + NIX - 2, iold))
            _when(j + 2 < TOT, lambda: gather_start(j + 2, rnext, inext))
        pltpu.make_async_copy(h_hbm.at[pl.ds(0, CHUNK)]
                              if not indirect else
                              h_hbm.at[sidx2.at[icur]],
                              rbs.at[rcur], semg.at[rcur]).wait()
        pltpu.make_async_copy(dst_hbm.at[pl.ds(0, CHUNK)], didx2.at[icur],
                              semdi.at[icur]).wait()
        pltpu.async_copy(rbs.at[rcur], sacc.at[didx2.at[icur]],
                         sems.at[rcur], add=True)

    for t in range(NIX):
        idx_start(t, t)
    for t in range(2):
        gather_start(t, t, t)

    ngroups = TOT // NIX
    def body(g, carry):
        j0 = g * NIX
        for t in range(NIX):
            sub(j0 + t, t)
        return carry

    lax.fori_loop(0, ngroups, body, 0, unroll=False)
    for t in range(TOT - ngroups * NIX):
        sub(ngroups * NIX + t, t)

    scatter_wait((TOT - 2) % NRB, (TOT - 2) % NIX)
    scatter_wait((TOT - 1) % NRB, (TOT - 1) % NIX)

    plsc.subcore_barrier()
    pltpu.sync_copy(sacc.at[pl.ds(r0, RPT)], out_hbm.at[c, pl.ds(r0, RPT)])
    plsc.subcore_barrier()


def _sc_attr_body(ea0, d0, ea1, d1, z16, out0, out1,
                  sidx2, didx2, rbs, sacc, semg, sems, semsi, semdi):
    c = lax.axis_index("c")
    s = lax.axis_index("s")
    wid = s * NC + c
    _pipelined_phase(False, ea0, None, d0, out0, c, s, wid, sidx2, didx2,
                     rbs, sacc, z16, semg, sems, semsi, semdi)
    _pipelined_phase(False, ea1, None, d1, out1, c, s, wid, sidx2, didx2,
                     rbs, sacc, z16, semg, sems, semsi, semdi)


def _sc_layer_body(hu, hi, su2i, du2i, si2u, di2u, zrows,
                   out_i, out_u, sidx2, didx2, rbs, sacc,
                   semg, sems, semsi, semdi):
    c = lax.axis_index("c")
    s = lax.axis_index("s")
    wid = s * NC + c
    _pipelined_phase(True, hu, su2i, du2i, out_i, c, s, wid, sidx2, didx2,
                     rbs, sacc, zrows, semg, sems, semsi, semdi)
    _pipelined_phase(True, hi, si2u, di2u, out_u, c, s, wid, sidx2, didx2,
                     rbs, sacc, zrows, semg, sems, semsi, semdi)


_sc_scratch = [
    pltpu.VMEM((NIX, CHUNK), jnp.int32),
    pltpu.VMEM((NIX, CHUNK), jnp.int32),
    pltpu.VMEM((NRB, CHUNK, DD), jnp.float32),
    pltpu.VMEM_SHARED((NPAD, DD), jnp.float32),
    pltpu.SemaphoreType.DMA((NRB,)),
    pltpu.SemaphoreType.DMA((NRB,)),
    pltpu.SemaphoreType.DMA((NIX,)),
    pltpu.SemaphoreType.DMA((NIX,)),
]

_sc_attr = pl.kernel(
    _sc_attr_body,
    out_type=(
        jax.ShapeDtypeStruct((NC, NPAD, DD), jnp.float32),
        jax.ShapeDtypeStruct((NC, NPAD, DD), jnp.float32),
    ),
    mesh=_MESH,
    scratch_types=list(_sc_scratch),
)

_sc_layer = pl.kernel(
    _sc_layer_body,
    out_type=(
        jax.ShapeDtypeStruct((NC, NPAD, DD), jnp.float32),
        jax.ShapeDtypeStruct((NC, NPAD, DD), jnp.float32),
    ),
    mesh=_MESH,
    scratch_types=list(_sc_scratch),
)


def _tc_layer_kern(pu, pi, au, ai, hu, hi, weu, wei,
                   wlu, wru, wli, wri, bb, gam, bet, ou, oi):
    def side(p, a, h, we, wl, wr, row):
        ssum = p[0] + p[1]
        a16 = (a[0] + a[1])[:, :16]
        cnt = a16[:, 15:16]
        se = jnp.dot(a16, we[...], preferred_element_type=jnp.float32)
        agg = (ssum + se) / jnp.maximum(cnt, 1.0)
        out = (jnp.dot(agg, wl[...], preferred_element_type=jnp.float32)
               + jnp.dot(h[...], wr[...], preferred_element_type=jnp.float32)
               + bb[row:row + 1])
        m = jnp.mean(out, axis=-1, keepdims=True)
        var = jnp.mean((out - m) ** 2, axis=-1, keepdims=True)
        return (out - m) * lax.rsqrt(var + 1e-5) * gam[row:row + 1] + bet[row:row + 1]

    ou[...] = side(pu, au, hu, weu, wlu, wru, 0)
    oi[...] = side(pi, ai, hi, wei, wli, wri, 1)


def _tc_layer(pu, pi, au, ai, hu, hi, weu, wei, wlu, wru, wli, wri, bb, gam, bet):
    bt = 1000
    grid = (NUSER // bt,)
    full2 = lambda shape: pl.BlockSpec(shape, lambda b: (0, 0))
    out = pl.pallas_call(
        _tc_layer_kern,
        grid=grid,
        in_specs=[
            pl.BlockSpec((NC, bt, DD), lambda b: (0, b, 0)),
            pl.BlockSpec((NC, bt, DD), lambda b: (0, b, 0)),
            pl.BlockSpec((NC, bt, DD), lambda b: (0, b, 0)),
            pl.BlockSpec((NC, bt, DD), lambda b: (0, b, 0)),
            pl.BlockSpec((bt, DD), lambda b: (b, 0)),
            pl.BlockSpec((bt, DD), lambda b: (b, 0)),
            full2((16, DD)), full2((16, DD)),
            full2((DD, DD)), full2((DD, DD)), full2((DD, DD)), full2((DD, DD)),
            full2((2, DD)), full2((2, DD)), full2((2, DD)),
        ],
        out_specs=[
            pl.BlockSpec((bt, DD), lambda b: (b, 0)),
            pl.BlockSpec((bt, DD), lambda b: (b, 0)),
        ],
        out_shape=[
            jax.ShapeDtypeStruct((NUSER, DD), jnp.float32),
            jax.ShapeDtypeStruct((NITEM, DD), jnp.float32),
        ],
    )(pu, pi, au, ai, hu, hi, weu, wei, wlu, wru, wli, wri, bb, gam, bet)
    return out


def kernel(x_user, x_item, edge_index_u2i, edge_index_i2u,
           edge_attr_u2i, edge_attr_i2u, Wl, bl, Wr, br, We, be, gamma, beta):
    f32 = jnp.float32
    su2i = edge_index_u2i[0].astype(jnp.int32)
    du2i = edge_index_u2i[1].astype(jnp.int32)
    si2u = edge_index_i2u[0].astype(jnp.int32)
    di2u = edge_index_i2u[1].astype(jnp.int32)

    def pad128(ea):
        z = jnp.zeros((NE, 6), f32)
        o = jnp.ones((NE, 1), f32)
        z2 = jnp.zeros((NE, DD - 16), f32)
        return jnp.concatenate([ea.astype(f32), z, o, z2], axis=1)

    ea0 = pad128(edge_attr_u2i)
    ea1 = pad128(edge_attr_i2u)

    def padWe(e):
        return jnp.concatenate(
            [We[e].astype(f32), jnp.zeros((6, DD), f32), be[e][None].astype(f32)],
            axis=0)

    we_i = padWe(0)
    we_u = padWe(1)

    z16 = jnp.zeros((RPT, DD), f32)
    zrows = jnp.zeros((RPT, DD), f32)

    ai16, au16 = _sc_attr(ea0, du2i, ea1, di2u, z16)

    hu = x_user.astype(f32)
    hi = x_item.astype(f32)
    gam = gamma.astype(f32)
    bet = beta.astype(f32)

    for l in range(NLAYERS):
        pi, pu = _sc_layer(hu, hi, su2i, du2i, si2u, di2u, zrows)
        bb = jnp.stack([bl[1, l] + br[1, l], bl[0, l] + br[0, l]], axis=0).astype(f32)
        hu, hi = _tc_layer(pu, pi, au16, ai16, hu, hi, we_u, we_i,
                           Wl[1, l], Wr[1, l], Wl[0, l], Wr[0, l], bb, gam, bet)
    return jnp.concatenate([hu, hi], axis=0)
